# baseline (speedup 1.0000x reference)
import functools

import jax
import jax.numpy as jnp
from jax import lax
from jax.experimental import pallas as pl
from jax.experimental.pallas import tpu as pltpu
from jax.experimental.pallas import tpu_sc as plsc

NUM_CODES = 8192
DIM = 256
NUM_TOKENS = 8192

TB = 8192
CB = 256
NI = NUM_TOKENS // TB
NJ = NUM_CODES // CB

SC_CORES = 2
SC_SUBCORES = 16
SC_WORKERS = SC_CORES * SC_SUBCORES
TOK_PER_W = NUM_TOKENS // SC_WORKERS
GCHUNK = 128
N_GCHUNK = TOK_PER_W // GCHUNK


def _argmin_body(f2_ref, e_ref, fn_ref, en_ref, lane_ref, o_ref, smax, sarg):
    j = pl.program_id(1)

    m2 = lax.dot_general(
        f2_ref[...], e_ref[...],
        dimension_numbers=(((1,), (0,)), ((), ())),
        preferred_element_type=jnp.float32,
    )
    neg = (m2 - fn_ref[...]) - en_ref[...]
    lanes = jnp.broadcast_to(lane_ref[...], (TB, CB))

    @pl.when(j == 0)
    def _init():
        smax[...] = neg
        sarg[...] = lanes

    @pl.when(j > 0)
    def _update():
        better = neg > smax[...]
        sarg[...] = jnp.where(better, lanes, sarg[...])
        smax[...] = jnp.maximum(neg, smax[...])

    @pl.when(j == NJ - 1)
    def _flush():
        v = smax[...]
        rowmax = jnp.max(v, axis=1, keepdims=True)
        cand = jnp.min(
            jnp.where(v == rowmax, sarg[...], jnp.float32(2**30)),
            axis=1, keepdims=True,
        )
        o_ref[...] = cand.astype(jnp.int32)


def _tc_argmin(flat2, emb, fnorm, enorm, lanes):
    return pl.pallas_call(
        _argmin_body,
        grid=(NI, NJ),
        in_specs=[
            pl.BlockSpec((TB, DIM), lambda i, j: (i, 0)),
            pl.BlockSpec((DIM, CB), lambda i, j: (0, j)),
            pl.BlockSpec((TB, 1), lambda i, j: (i, 0)),
            pl.BlockSpec((1, CB), lambda i, j: (0, j)),
            pl.BlockSpec((1, CB), lambda i, j: (0, j)),
        ],
        out_specs=pl.BlockSpec((TB, 1), lambda i, j: (i, 0)),
        out_shape=jax.ShapeDtypeStruct((NUM_TOKENS, 1), jnp.int32),
        scratch_shapes=[
            pltpu.VMEM((TB, CB), jnp.float32),
            pltpu.VMEM((TB, CB), jnp.float32),
        ],
        compiler_params=pltpu.CompilerParams(
            dimension_semantics=("parallel", "arbitrary"),
        ),
    )(flat2, emb, fnorm, enorm, lanes)


def _sc_gather(et, idx):
    mesh = plsc.VectorSubcoreMesh(
        core_axis_name="c", subcore_axis_name="s"
    )

    @functools.partial(
        pl.kernel,
        mesh=mesh,
        out_type=jax.ShapeDtypeStruct((NUM_TOKENS, DIM), jnp.float32),
        scratch_types=[
            pltpu.VMEM((GCHUNK,), jnp.int32),
            pltpu.VMEM((GCHUNK, DIM), jnp.float32),
            pltpu.SemaphoreType.DMA,
        ],
    )
    def gather_k(et_hbm, idx_hbm, out_hbm, idx_v, rows_v, sem):
        wid = lax.axis_index("s") * SC_CORES + lax.axis_index("c")
        base = wid * TOK_PER_W
        for h in range(N_GCHUNK):
            off = base + h * GCHUNK
            pltpu.sync_copy(idx_hbm.at[pl.ds(off, GCHUNK)], idx_v)
            pltpu.async_copy(et_hbm.at[idx_v], rows_v, sem).wait()
            pltpu.sync_copy(rows_v, out_hbm.at[pl.ds(off, GCHUNK)])

    return gather_k(et, idx)


def kernel(inputs, embeddings):
    shp = inputs.shape
    flat = inputs.reshape(-1, DIM)
    fnorm = jnp.sum(flat**2, axis=1, keepdims=True)
    enorm = jnp.sum(embeddings**2, axis=0, keepdims=True)
    lanes = jnp.arange(NUM_CODES, dtype=jnp.float32).reshape(1, NUM_CODES)
    idx = _tc_argmin(flat + flat, embeddings, fnorm, enorm, lanes).reshape(-1)
    et = embeddings.T
    return _sc_gather(et, idx).reshape(shp)


# speedup vs baseline: 1.2922x; 1.2922x over previous
import functools

import jax
import jax.numpy as jnp
from jax import lax
from jax.experimental import pallas as pl
from jax.experimental.pallas import tpu as pltpu
from jax.experimental.pallas import tpu_sc as plsc

NUM_CODES = 8192
DIM = 256
NUM_TOKENS = 8192

TB = 4096
CB = 2048
NI = NUM_TOKENS // TB
NJ = NUM_CODES // CB

SC_CORES = 2
SC_SUBCORES = 16
SC_WORKERS = SC_CORES * SC_SUBCORES
TOK_PER_W = NUM_TOKENS // SC_WORKERS
GCHUNK = 128
N_GCHUNK = TOK_PER_W // GCHUNK


def _argmin_body(f2_ref, e_ref, fn_ref, en_ref, lane_ref, o_ref, smax, sarg):
    j = pl.program_id(1)

    @pl.when(j == 0)
    def _init():
        smax[...] = jnp.full((TB, 1), -jnp.inf, jnp.float32)
        sarg[...] = jnp.zeros((TB, 1), jnp.float32)

    m2 = lax.dot_general(
        f2_ref[...], e_ref[...],
        dimension_numbers=(((1,), (0,)), ((), ())),
        preferred_element_type=jnp.float32,
    )
    neg = (m2 - fn_ref[...]) - en_ref[...]
    rowmax = jnp.max(neg, axis=1, keepdims=True)
    cand = jnp.min(
        jnp.where(neg == rowmax, lane_ref[...], jnp.float32(2**30)),
        axis=1, keepdims=True,
    )
    better = rowmax > smax[...]
    sarg[...] = jnp.where(better, cand, sarg[...])
    smax[...] = jnp.where(better, rowmax, smax[...])

    @pl.when(j == NJ - 1)
    def _flush():
        o_ref[...] = sarg[...].astype(jnp.int32)


def _tc_argmin(flat2, emb, fnorm, enorm, lanes):
    return pl.pallas_call(
        _argmin_body,
        grid=(NI, NJ),
        in_specs=[
            pl.BlockSpec((TB, DIM), lambda i, j: (i, 0)),
            pl.BlockSpec((DIM, CB), lambda i, j: (0, j)),
            pl.BlockSpec((TB, 1), lambda i, j: (i, 0)),
            pl.BlockSpec((1, CB), lambda i, j: (0, j)),
            pl.BlockSpec((1, CB), lambda i, j: (0, j)),
        ],
        out_specs=pl.BlockSpec((TB, 1), lambda i, j: (i, 0)),
        out_shape=jax.ShapeDtypeStruct((NUM_TOKENS, 1), jnp.int32),
        scratch_shapes=[
            pltpu.VMEM((TB, 1), jnp.float32),
            pltpu.VMEM((TB, 1), jnp.float32),
        ],
        compiler_params=pltpu.CompilerParams(
            dimension_semantics=("parallel", "arbitrary"),
        ),
    )(flat2, emb, fnorm, enorm, lanes)


def _sc_gather(et, idx):
    mesh = plsc.VectorSubcoreMesh(
        core_axis_name="c", subcore_axis_name="s"
    )

    @functools.partial(
        pl.kernel,
        mesh=mesh,
        out_type=jax.ShapeDtypeStruct((NUM_TOKENS, DIM), jnp.float32),
        scratch_types=[
            pltpu.VMEM((GCHUNK,), jnp.int32),
            pltpu.VMEM((GCHUNK, DIM), jnp.float32),
            pltpu.SemaphoreType.DMA,
        ],
    )
    def gather_k(et_hbm, idx_hbm, out_hbm, idx_v, rows_v, sem):
        wid = lax.axis_index("s") * SC_CORES + lax.axis_index("c")
        base = wid * TOK_PER_W
        for h in range(N_GCHUNK):
            off = base + h * GCHUNK
            pltpu.sync_copy(idx_hbm.at[pl.ds(off, GCHUNK)], idx_v)
            pltpu.async_copy(et_hbm.at[idx_v], rows_v, sem).wait()
            pltpu.sync_copy(rows_v, out_hbm.at[pl.ds(off, GCHUNK)])

    return gather_k(et, idx)


def kernel(inputs, embeddings):
    shp = inputs.shape
    flat = inputs.reshape(-1, DIM)
    fnorm = jnp.sum(flat**2, axis=1, keepdims=True)
    enorm = jnp.sum(embeddings**2, axis=0, keepdims=True)
    lanes = jnp.arange(NUM_CODES, dtype=jnp.float32).reshape(1, NUM_CODES)
    idx = _tc_argmin(flat + flat, embeddings, fnorm, enorm, lanes).reshape(-1)
    et = embeddings.T
    return _sc_gather(et, idx).reshape(shp)
